# baseline (speedup 1.0000x reference)
import math

import jax
import jax.numpy as jnp
from jax.experimental import pallas as pl
from jax.experimental.pallas import tpu as pltpu

N = 116
NP = 128
G = 8
HID = [32, 64, 128, 256, 512]
HEADS = [8, 4, 2, 1]
NEG = -1e9

_f32 = jnp.float32


def _dot(a, b):
    return jnp.dot(a, b, preferred_element_type=_f32)


def _dot_t(a, b):
    return jax.lax.dot_general(a, b, (((1,), (1,)), ((), ())),
                               preferred_element_type=_f32)


def _gt_layer(h, penalties, rowhas, ones, Wq, Wk, Wv, Wr, b, heads):
    d_out = Wq.shape[1]
    hd = d_out // heads
    scale = 1.0 / math.sqrt(hd)
    q = _dot(h, Wq) * scale
    k = _dot(h, Wk)
    v = _dot(h, Wv)
    r = _dot(h, Wr)

    aligned = hd % 128 == 0
    if heads > 1 and not aligned:
        lane = jax.lax.broadcasted_iota(jnp.int32, (1, d_out), 1) // hd
        qts = [jnp.where(lane == hh, q, 0.0) for hh in range(heads)]
        vts = [jnp.where(lane == hh, v, 0.0) for hh in range(heads)]

    outs = []
    for g in range(G):
        sl = slice(g * NP, (g + 1) * NP)
        kg = k[sl]
        acc = None
        for hh in range(heads):
            if heads == 1:
                logits = _dot_t(q[sl], kg)
            elif aligned:
                hsl = slice(hh * hd, (hh + 1) * hd)
                logits = _dot_t(q[sl][:, hsl], kg[:, hsl])
            else:
                logits = _dot_t(qts[hh][sl], kg)
            logits = logits + penalties[g]
            m = jnp.max(logits, axis=1, keepdims=True)
            e = jnp.exp(logits - m)
            s = _dot(e, ones)
            f = rowhas[g] / s
            if heads == 1:
                o = _dot(e, v[sl]) * f
            elif aligned:
                o = jnp.pad(_dot(e, v[sl][:, hsl]) * f,
                            ((0, 0), (hh * hd, d_out - (hh + 1) * hd)))
            else:
                o = _dot(e, vts[hh][sl]) * f
            acc = o if acc is None else acc + o
        outs.append(acc)
    out = jnp.concatenate(outs, axis=0)
    return jnp.maximum(out + r + b, 0.0)


def _fwd_kernel(x_ref, adjT_ref, W_in_ref, b_in_ref,
                Wq1, Wk1, Wv1, Wr1, b1,
                Wq2, Wk2, Wv2, Wr2, b2,
                Wq3, Wk3, Wv3, Wr3, b3,
                Wq4, Wk4, Wv4, Wr4, b4,
                Wa_ref, Wfh_ref, Wfa_ref, bf_ref,
                att_ref, logit_ref):
    x = x_ref[...].reshape(G * NP, NP)
    ones = jnp.ones((NP, 1), _f32)

    penalties, rowhas = [], []
    for g in range(G):
        mf = (adjT_ref[g] > 0.0).astype(_f32)
        penalties.append((mf - 1.0) * 1e9)
        rowhas.append((_dot(mf, ones) > 0.0).astype(_f32))

    h = _dot(x, W_in_ref[...]) + b_in_ref[...]
    h = _gt_layer(h, penalties, rowhas, ones, Wq1[...], Wk1[...], Wv1[...], Wr1[...], b1[...], 8)
    h = _gt_layer(h, penalties, rowhas, ones, Wq2[...], Wk2[...], Wv2[...], Wr2[...], b2[...], 4)
    h = _gt_layer(h, penalties, rowhas, ones, Wq3[...], Wk3[...], Wv3[...], Wr3[...], b3[...], 2)
    h = _gt_layer(h, penalties, rowhas, ones, Wq4[...], Wk4[...], Wv4[...], Wr4[...], b4[...], 1)

    hw = _dot(h, Wa_ref[...])
    colpen = jnp.where(
        jax.lax.broadcasted_iota(jnp.int32, (NP, NP), 1) < N, 0.0, NEG)
    rowv = jnp.where(
        jax.lax.broadcasted_iota(jnp.int32, (1, NP), 1) < N, 1.0 / N, 0.0)
    fscale = 1.0 / math.sqrt(HID[4])
    for g in range(G):
        sl = slice(g * NP, (g + 1) * NP)
        hg = h[sl]
        scores = _dot_t(hw[sl], hg) * fscale + colpen
        m = jnp.max(scores, axis=1, keepdims=True)
        e = jnp.exp(scores - m)
        s = _dot(e, ones)
        att = e / s
        att_ref[g] = att
        pooled_h = _dot(rowv, hg)
        pooled_a = _dot(rowv, att)
        logit_ref[g] = (_dot(pooled_h, Wfh_ref[...])
                        + _dot(pooled_a, Wfa_ref[...]) + bf_ref[...])


def kernel(x, adj, W_in, b_in, Wq1, Wk1, Wv1, Wr1, b1, Wq2, Wk2, Wv2, Wr2, b2,
           Wq3, Wk3, Wv3, Wr3, b3, Wq4, Wk4, Wv4, Wr4, b4, Wa, Wf, bf):
    B = x.shape[0]

    xp = jnp.pad(x, ((0, 0), (0, NP - N), (0, NP - N)))
    adjT = jnp.pad(jnp.swapaxes(adj, 1, 2), ((0, 0), (0, NP - N), (0, NP - N)))
    W_in_p = jnp.pad(W_in, ((0, NP - N), (0, 0)))
    Wfh = Wf[:HID[4]]
    Wfa = jnp.pad(Wf[HID[4]:], ((0, NP - N), (0, 0)))
    b_in2 = b_in.reshape(1, -1)
    bs = [b1.reshape(1, -1), b2.reshape(1, -1), b3.reshape(1, -1), b4.reshape(1, -1)]
    bf2 = bf.reshape(1, -1)

    def wspec(a):
        return pl.BlockSpec(a.shape, lambda b: (0,) * a.ndim)

    layer_ws = [Wq1, Wk1, Wv1, Wr1, bs[0],
                Wq2, Wk2, Wv2, Wr2, bs[1],
                Wq3, Wk3, Wv3, Wr3, bs[2],
                Wq4, Wk4, Wv4, Wr4, bs[3]]

    in_specs = [
        pl.BlockSpec((G, NP, NP), lambda b: (b, 0, 0)),
        pl.BlockSpec((G, NP, NP), lambda b: (b, 0, 0)),
        wspec(W_in_p), wspec(b_in2),
    ] + [wspec(w) for w in layer_ws] + [
        wspec(Wa), wspec(Wfh), wspec(Wfa), wspec(bf2),
    ]

    out_shapes = (
        jax.ShapeDtypeStruct((B, NP, NP), _f32),
        jax.ShapeDtypeStruct((B, 1, 2), _f32),
    )
    out_specs = (
        pl.BlockSpec((G, NP, NP), lambda b: (b, 0, 0)),
        pl.BlockSpec((G, 1, 2), lambda b: (b, 0, 0)),
    )

    att_p, logit3 = pl.pallas_call(
        _fwd_kernel,
        grid=(B // G,),
        in_specs=in_specs,
        out_specs=out_specs,
        out_shape=out_shapes,
        compiler_params=pltpu.CompilerParams(
            dimension_semantics=("parallel",)),
    )(xp, adjT, W_in_p, b_in2, *layer_ws, Wa, Wfh, Wfa, bf2)

    attention = att_p[:, :N, :N]
    logit = logit3[:, 0, :]
    return (attention, logit)


# speedup vs baseline: 1.0767x; 1.0767x over previous
import math

import jax
import jax.numpy as jnp
from jax.experimental import pallas as pl
from jax.experimental.pallas import tpu as pltpu

N = 116
NP = 128
G = 8
HID = [32, 64, 128, 256, 512]
NEG = -1e9

_f32 = jnp.float32
_bf16 = jnp.bfloat16


def _dot(a, b):
    return jnp.dot(a, b, preferred_element_type=_f32)


def _dot_t(a, b):
    return jax.lax.dot_general(a, b, (((1,), (1,)), ((), ())),
                               preferred_element_type=_f32)


def _gt_layer(hb, penalties, rowhas, ones, Wq, Wk, Wv, Wr, b, heads):
    d_out = Wq.shape[1]
    hd = d_out // heads
    scale = 1.0 / math.sqrt(hd)
    q = (_dot(hb, Wq) * scale).astype(_bf16)
    k = _dot(hb, Wk).astype(_bf16)
    v = _dot(hb, Wv).astype(_bf16)
    r = _dot(hb, Wr)

    aligned = hd % 128 == 0
    if heads > 1 and not aligned:
        lane = jax.lax.broadcasted_iota(jnp.int32, (1, d_out), 1) // hd
        zb = jnp.zeros((), _bf16)
        qts = [jnp.where(lane == hh, q, zb) for hh in range(heads)]
        vts = [jnp.where(lane == hh, v, zb) for hh in range(heads)]

    outs = []
    for g in range(G):
        sl = slice(g * NP, (g + 1) * NP)
        kg = k[sl]
        acc = None
        for hh in range(heads):
            if heads == 1:
                logits = _dot_t(q[sl], kg)
            elif aligned:
                hsl = slice(hh * hd, (hh + 1) * hd)
                logits = _dot_t(q[sl][:, hsl], kg[:, hsl])
            else:
                logits = _dot_t(qts[hh][sl], kg)
            logits = logits + penalties[g]
            m = jnp.max(logits, axis=1, keepdims=True)
            e = jnp.exp(logits - m).astype(_bf16)
            s = _dot(e, ones)
            f = rowhas[g] * (1.0 / s)
            if heads == 1:
                o = _dot(e, v[sl]) * f
            elif aligned:
                o = jnp.pad(_dot(e, v[sl][:, hsl]) * f,
                            ((0, 0), (hh * hd, d_out - (hh + 1) * hd)))
            else:
                o = _dot(e, vts[hh][sl]) * f
            acc = o if acc is None else acc + o
        outs.append(acc)
    out = jnp.concatenate(outs, axis=0)
    return jnp.maximum(out + r + b, 0.0).astype(_bf16)


def _fwd_kernel(x_ref, adjT_ref, W_in_ref, b_in_ref,
                Wq1, Wk1, Wv1, Wr1, b1,
                Wq2, Wk2, Wv2, Wr2, b2,
                Wq3, Wk3, Wv3, Wr3, b3,
                Wq4, Wk4, Wv4, Wr4, b4,
                Wa_ref, Wfh_ref, Wfa_ref, bf_ref,
                att_ref, logit_ref):
    x = x_ref[...].reshape(G * NP, NP)
    ones = jnp.ones((NP, 1), _bf16)

    penalties, rowhas = [], []
    for g in range(G):
        mf = (adjT_ref[g] > 0.0).astype(_f32)
        penalties.append((mf - 1.0) * 1e9)
        rowhas.append((_dot(mf.astype(_bf16), ones) > 0.0).astype(_f32))

    h = (_dot(x.astype(_bf16), W_in_ref[...]) + b_in_ref[...]).astype(_bf16)
    h = _gt_layer(h, penalties, rowhas, ones, Wq1[...], Wk1[...], Wv1[...], Wr1[...], b1[...], 8)
    h = _gt_layer(h, penalties, rowhas, ones, Wq2[...], Wk2[...], Wv2[...], Wr2[...], b2[...], 4)
    h = _gt_layer(h, penalties, rowhas, ones, Wq3[...], Wk3[...], Wv3[...], Wr3[...], b3[...], 2)
    h = _gt_layer(h, penalties, rowhas, ones, Wq4[...], Wk4[...], Wv4[...], Wr4[...], b4[...], 1)

    hw = _dot(h, Wa_ref[...]).astype(_bf16)
    colpen = jnp.where(
        jax.lax.broadcasted_iota(jnp.int32, (NP, NP), 1) < N, 0.0, NEG)
    rowv = jnp.where(
        jax.lax.broadcasted_iota(jnp.int32, (1, NP), 1) < N, 1.0 / N, 0.0)
    rowvb = rowv.astype(_bf16)
    fscale = 1.0 / math.sqrt(HID[4])
    for g in range(G):
        sl = slice(g * NP, (g + 1) * NP)
        hg = h[sl]
        scores = _dot_t(hw[sl], hg) * fscale + colpen
        m = jnp.max(scores, axis=1, keepdims=True)
        e = jnp.exp(scores - m)
        eb = e.astype(_bf16)
        s = _dot(eb, ones)
        att = e * (1.0 / s)
        att_ref[g] = att
        pooled_h = _dot(rowvb, hg)
        pooled_a = _dot(rowv, att)
        logit_ref[g] = (_dot(pooled_h, Wfh_ref[...])
                        + _dot(pooled_a, Wfa_ref[...]) + bf_ref[...])


def kernel(x, adj, W_in, b_in, Wq1, Wk1, Wv1, Wr1, b1, Wq2, Wk2, Wv2, Wr2, b2,
           Wq3, Wk3, Wv3, Wr3, b3, Wq4, Wk4, Wv4, Wr4, b4, Wa, Wf, bf):
    B = x.shape[0]

    xp = jnp.pad(x, ((0, 0), (0, NP - N), (0, NP - N)))
    adjT = jnp.pad(jnp.swapaxes(adj, 1, 2), ((0, 0), (0, NP - N), (0, NP - N)))
    W_in_p = jnp.pad(W_in, ((0, NP - N), (0, 0))).astype(_bf16)
    Wfh = Wf[:HID[4]]
    Wfa = jnp.pad(Wf[HID[4]:], ((0, NP - N), (0, 0)))
    b_in2 = b_in.reshape(1, -1)
    bs = [b1.reshape(1, -1), b2.reshape(1, -1), b3.reshape(1, -1), b4.reshape(1, -1)]
    bf2 = bf.reshape(1, -1)

    def wspec(a):
        return pl.BlockSpec(a.shape, lambda b: (0,) * a.ndim)

    bw = lambda w: w.astype(_bf16)
    layer_ws = [bw(Wq1), bw(Wk1), bw(Wv1), bw(Wr1), bs[0],
                bw(Wq2), bw(Wk2), bw(Wv2), bw(Wr2), bs[1],
                bw(Wq3), bw(Wk3), bw(Wv3), bw(Wr3), bs[2],
                bw(Wq4), bw(Wk4), bw(Wv4), bw(Wr4), bs[3]]

    in_specs = [
        pl.BlockSpec((G, NP, NP), lambda b: (b, 0, 0)),
        pl.BlockSpec((G, NP, NP), lambda b: (b, 0, 0)),
        wspec(W_in_p), wspec(b_in2),
    ] + [wspec(w) for w in layer_ws] + [
        wspec(Wa), wspec(Wfh), wspec(Wfa), wspec(bf2),
    ]

    out_shapes = (
        jax.ShapeDtypeStruct((B, NP, NP), _f32),
        jax.ShapeDtypeStruct((B, 1, 2), _f32),
    )
    out_specs = (
        pl.BlockSpec((G, NP, NP), lambda b: (b, 0, 0)),
        pl.BlockSpec((G, 1, 2), lambda b: (b, 0, 0)),
    )

    att_p, logit3 = pl.pallas_call(
        _fwd_kernel,
        grid=(B // G,),
        in_specs=in_specs,
        out_specs=out_specs,
        out_shape=out_shapes,
        compiler_params=pltpu.CompilerParams(
            dimension_semantics=("parallel",)),
    )(xp, adjT, W_in_p, b_in2, *layer_ws, bw(Wa), Wfh, Wfa, bf2)

    attention = att_p[:, :N, :N]
    logit = logit3[:, 0, :]
    return (attention, logit)
